# baseline (speedup 1.0000x reference)
import jax
import jax.numpy as jnp
from jax import lax
from jax.experimental import pallas as pl
from jax.experimental.pallas import tpu as pltpu
from jax.experimental.pallas import tpu_sc as plsc

V = 1000000
D = 64
BLOCK = 4096
N_BLOCKS = (V * D + BLOCK - 1) // BLOCK
AM_PAD = 16384

NC = 2
NS = 16
NW = NC * NS
L = 16
NSPAN = D // L

XROWS = 16384
XCOLS = 50
B_TOTAL = XROWS * XCOLS
PER_TILE = B_TOTAL // NW
XR_TILE = XROWS // NW
CHUNK = 128
NCHUNK = PER_TILE // CHUNK
NBUF = 4
NRING = 8
RB = 4
NBATCH = CHUNK // RB

NEWTON_ITERS = 2


def _vf(x):
    return jnp.full((L,), x, jnp.float32)


def _vi(x):
    return jnp.full((L,), x, jnp.int32)


def _body(x_hbm, w_hbm, am_hbm, code_hbm, lnw_hbm, lnb_hbm, out_hbm, *rest):
    x2_v, ring_v, am_v, code_v, lnw_v, lnb_v, crep_v, amc_v = rest[:8]
    rows = rest[8:8 + NBUF]
    outs = rest[8 + NBUF:8 + 2 * NBUF]
    gsems = rest[8 + 2 * NBUF:8 + 3 * NBUF]
    osems = rest[8 + 3 * NBUF:8 + 4 * NBUF]

    c = lax.axis_index("c")
    s = lax.axis_index("s")
    wid = s * NC + c
    xr_base = wid * XR_TILE
    out_base = wid * PER_TILE

    pltpu.sync_copy(x_hbm.at[pl.ds(xr_base, XR_TILE)], x2_v)
    pltpu.sync_copy(am_hbm, am_v)
    pltpu.sync_copy(code_hbm, code_v)
    pltpu.sync_copy(lnw_hbm, lnw_v)
    pltpu.sync_copy(lnb_hbm, lnb_v)

    iota = lax.iota(jnp.int32, L)
    zero16 = _vi(0)

    @pl.loop(0, 256, unroll=4)
    def crep(ci):
        bc = plsc.load_gather(code_v, [jnp.full((L,), ci, jnp.int32)])
        crep_v[pl.ds(ci * L, L)] = bc

    def build_chunk(jj):
        base = jj * CHUNK
        slot = lax.bitwise_and(jj, NRING - 1)
        for g in range(CHUNK // L):
            flat = base + g * L + iota
            q = lax.shift_right_logical(flat * _vi(5243), _vi(18))
            rem = flat - q * _vi(XCOLS)
            val = plsc.load_gather(x2_v, [q, rem])
            ring_v[slot, pl.ds(g * L, L)] = val

    lnw_regs = tuple(lnw_v[pl.ds(sp * L, L)] for sp in range(NSPAN))
    lnb_regs = tuple(lnb_v[pl.ds(sp * L, L)] for sp in range(NSPAN))
    inv_d = _vf(1.0 / D)
    eps = _vf(1e-5)
    perms = tuple(lax.bitwise_xor(iota, _vi(k)) for k in (1, 2, 4, 8))

    def compute_chunk(j, rows_v, out_v):
        @pl.loop(0, NBATCH)
        def batch(b2):
            rb = b2 * RB
            rrs = [rb + r for r in range(RB)]
            cs = [[rows_v[rr, pl.ds(sp * L, L)] for sp in range(NSPAN)]
                  for rr in rrs]
            amv = [plsc.load_gather(amc_v, [jnp.full((L,), rr, jnp.int32)])
                   for rr in rrs]
            ci = [[lax.shift_left(cs[r][sp], _vi(4)) + iota
                   for sp in range(NSPAN)] for r in range(RB)]
            u = [[plsc.load_gather(crep_v, [ci[r][sp]])
                  for sp in range(NSPAN)] for r in range(RB)]
            t01 = [u[r][0] + u[r][1] for r in range(RB)]
            t23 = [u[r][2] + u[r][3] for r in range(RB)]
            sq = [[u[r][sp] * u[r][sp] for sp in range(NSPAN)]
                  for r in range(RB)]
            q01 = [sq[r][0] + sq[r][1] for r in range(RB)]
            q23 = [sq[r][2] + sq[r][3] for r in range(RB)]
            vv = [t01[r] + t23[r] for r in range(RB)] + \
                 [q01[r] + q23[r] for r in range(RB)]
            for perm in perms:
                pv = [v.at[perm].get(mode="promise_in_bounds") for v in vv]
                vv = [v + p for v, p in zip(vv, pv)]
            mean = [sm * inv_d for sm in vv[:RB]]
            e2 = [sm * inv_d for sm in vv[RB:]]
            mm = [m * m for m in mean]
            var = [e - m2 for e, m2 in zip(e2, mm)]
            am2 = [a * a for a in amv]
            ve = [v * a2 + eps for v, a2 in zip(var, am2)]
            half = [v * _vf(0.5) for v in ve]
            iv = [plsc.bitcast(v, jnp.int32) for v in ve]
            iv = [_vi(0x5F3759DF) - lax.shift_right_logical(i2, _vi(1))
                  for i2 in iv]
            y = [plsc.bitcast(i2, jnp.float32) for i2 in iv]
            for _ in range(NEWTON_ITERS):
                yy = [yi * yi for yi in y]
                hyy = [h * t for h, t in zip(half, yy)]
                th = [_vf(1.5) - t for t in hyy]
                y = [yi * t for yi, t in zip(y, th)]
            scale = [a * yi for a, yi in zip(amv, y)]
            for sp in range(NSPAN):
                t = [u[r][sp] - mean[r] for r in range(RB)]
                t = [tt * scale[r] for r, tt in enumerate(t)]
                t = [tt * lnw_regs[sp] for tt in t]
                t = [tt + lnb_regs[sp] for tt in t]
                for r in range(RB):
                    out_v[rrs[r], pl.ds(sp * L, L)] = t[r]

    for b in range(NBUF):
        build_chunk(b)
        pltpu.async_copy(w_hbm.at[ring_v.at[b]], rows[b], gsems[b])

    @pl.loop(0, NCHUNK // NBUF)
    def tloop(t):
        for b in range(NBUF):
            j = t * NBUF + b
            jslot = lax.bitwise_and(j, NRING - 1)

            for g in range(CHUNK // L):
                idxv = ring_v[jslot, pl.ds(g * L, L)]
                amv = plsc.load_gather(
                    am_v, [lax.shift_right_logical(idxv, _vi(6))])
                amc_v[pl.ds(g * L, L)] = amv

            pltpu.make_async_copy(
                w_hbm.at[ring_v.at[jslot]], rows[b], gsems[b]).wait()

            @pl.when(t > 0)
            def _():
                pltpu.make_async_copy(
                    outs[b], out_hbm.at[pl.ds(out_base, CHUNK)],
                    osems[b]).wait()

            compute_chunk(j, rows[b], outs[b])
            pltpu.async_copy(
                outs[b], out_hbm.at[pl.ds(out_base + j * CHUNK, CHUNK)],
                osems[b])
            jp = jnp.minimum(j + NBUF, NCHUNK - 1)
            build_chunk(jp)
            pltpu.async_copy(
                w_hbm.at[ring_v.at[lax.bitwise_and(jp, NRING - 1)]],
                rows[b], gsems[b])

    for b in range(NBUF):
        pltpu.make_async_copy(
            w_hbm.at[ring_v.at[(NCHUNK - 1) % NRING]], rows[b],
            gsems[b]).wait()
        pltpu.make_async_copy(
            outs[b], out_hbm.at[pl.ds(out_base, CHUNK)], osems[b]).wait()


@jax.jit
def _run(x, weight, am_pad, code, ln_weight, ln_bias):
    mesh = plsc.VectorSubcoreMesh(core_axis_name="c", subcore_axis_name="s")
    scratch = [
        pltpu.VMEM((XR_TILE, XCOLS), jnp.int32),
        pltpu.VMEM((NRING, CHUNK), jnp.int32),
        pltpu.VMEM((AM_PAD,), jnp.float32),
        pltpu.VMEM((256,), jnp.float32),
        pltpu.VMEM((D,), jnp.float32),
        pltpu.VMEM((D,), jnp.float32),
        pltpu.VMEM((256 * L,), jnp.float32),
        pltpu.VMEM((CHUNK,), jnp.float32),
    ]
    scratch += [pltpu.VMEM((CHUNK, D), jnp.int32) for _ in range(NBUF)]
    scratch += [pltpu.VMEM((CHUNK, D), jnp.float32) for _ in range(NBUF)]
    scratch += [pltpu.SemaphoreType.DMA for _ in range(2 * NBUF)]
    return pl.kernel(
        _body,
        out_type=jax.ShapeDtypeStruct((B_TOTAL, D), jnp.float32),
        mesh=mesh,
        compiler_params=pltpu.CompilerParams(
            needs_layout_passes=False, use_tc_tiling_on_sc=False),
        scratch_types=scratch,
    )(x, weight, am_pad, code, ln_weight, ln_bias)


def kernel(x, weight, absmax, code, ln_weight, ln_bias):
    am_pad = jnp.concatenate(
        [absmax, jnp.zeros((AM_PAD - N_BLOCKS,), jnp.float32)])
    out = _run(x, weight, am_pad, code, ln_weight, ln_bias)
    return out.reshape(XROWS, XCOLS, D)
